# Initial kernel scaffold; baseline (speedup 1.0000x reference)
import jax
import jax.numpy as jnp
from jax.experimental import pallas as pl


def kernel(adj, x, W1, att_src1, att_dst1, b1, W2, att_src2, att_dst2, b2):
    raise NotImplementedError("write your pallas kernel here")


# speedup vs baseline: 32.5059x; 32.5059x over previous
import jax
import jax.numpy as jnp
from jax.experimental import pallas as pl
from jax.experimental.pallas import tpu as pltpu

_T = 1024
_HEADS = 8
_NHID = 16
_NCLASS = 16
_HP = 24
_SLOPE = 0.2
_PREC = jax.lax.Precision.HIGHEST


def _prep_kernel(n_ref, x_ref, w1_ref, asrc_m_ref, adst_m_ref,
                 ht_ref, asrc_ref, adstT_ref):
    k = pl.program_id(0)
    n = n_ref[0]
    xb = x_ref[...]
    xb = jnp.where(jnp.isnan(xb), jnp.zeros_like(xb), xb)
    h = jnp.dot(xb, w1_ref[...], preferred_element_type=jnp.float32,
                precision=_PREC)
    svalid = (jax.lax.broadcasted_iota(jnp.int32, (_T, 1), 0) + k * _T) < n
    asrc = jnp.dot(h, asrc_m_ref[...], preferred_element_type=jnp.float32,
                   precision=_PREC)
    asrc_ref[...] = jnp.where(svalid, asrc, 0.0)
    adst = jnp.dot(h, adst_m_ref[...], preferred_element_type=jnp.float32,
                   precision=_PREC)
    lvalid = svalid.T
    adstT_ref[...] = jnp.where(lvalid, adst.T, 0.0)
    zrow = jnp.zeros((1, _T), jnp.float32)
    for hh in range(_HEADS):
        blk = h[:, hh * _NHID:(hh + 1) * _NHID]
        ht_ref[hh, 0:_NHID, :] = jnp.where(lvalid, blk.T, 0.0)
        ht_ref[hh, _NHID:_NHID + 1, :] = jnp.ones((1, _T), jnp.float32)
        for pp in range(_NHID + 1, _HP):
            ht_ref[hh, pp:pp + 1, :] = zrow


def _masked_e(adjb, a_s, a_d, i, j, n):
    rows = jax.lax.broadcasted_iota(jnp.int32, (_T, _T), 0) + j * _T
    cols = jax.lax.broadcasted_iota(jnp.int32, (_T, _T), 1) + i * _T
    valid = (rows < n) & (cols < n)
    m = jnp.where(valid, adjb, 0.0) + (rows == cols).astype(jnp.float32)
    t = a_s + a_d
    t = jnp.maximum(t, _SLOPE * t)
    return jnp.exp(t) * m


def _layer1_kernel(n_ref, adj_ref, ht_ref, asrc_ref, adstT_ref, w2_ref,
                   b1_ref, as2m_ref, ad2m_ref,
                   h1_ref, ht2_ref, asrc2_ref, adst2T_ref, acc_ref):
    i = pl.program_id(0)
    j = pl.program_id(1)
    ns = pl.num_programs(1)
    n = n_ref[0]

    @pl.when(j == 0)
    def _():
        acc_ref[...] = jnp.zeros_like(acc_ref)

    adjb = adj_ref[...]
    for hh in range(_HEADS):
        a_s = asrc_ref[:, hh:hh + 1]
        a_d = adstT_ref[hh:hh + 1, :]
        e = _masked_e(adjb, a_s, a_d, i, j, n)
        acc_ref[hh] += jnp.dot(ht_ref[hh], e,
                               preferred_element_type=jnp.float32,
                               precision=_PREC)

    @pl.when(j == ns - 1)
    def _():
        acc = acc_ref[...]
        numer = acc[:, 0:_NHID, :]
        denom = acc[:, _NHID:_NHID + 1, :]
        norm = numer / denom
        parts = [norm[hh].T for hh in range(_HEADS)]
        h1 = jnp.concatenate(parts, axis=1) + b1_ref[0:1, :]
        h1_ref[...] = h1
        hr = jnp.maximum(h1, 0.0)
        h2f = jnp.dot(hr, w2_ref[...], preferred_element_type=jnp.float32,
                      precision=_PREC)
        ht2_ref[0:_NCLASS, :] = h2f.T
        ht2_ref[_NCLASS:_NCLASS + 1, :] = jnp.ones((1, _T), jnp.float32)
        for pp in range(_NCLASS + 1, _HP):
            ht2_ref[pp:pp + 1, :] = jnp.zeros((1, _T), jnp.float32)
        asrc2_ref[...] = jnp.dot(h2f, as2m_ref[...],
                                 preferred_element_type=jnp.float32,
                                 precision=_PREC)
        adst2T_ref[...] = jnp.dot(h2f, ad2m_ref[...],
                                  preferred_element_type=jnp.float32,
                                  precision=_PREC).T


def _layer2_kernel(n_ref, adj_ref, ht2_ref, asrc2_ref, adst2T_ref, b2_ref,
                   h2_ref, acc_ref):
    i = pl.program_id(0)
    j = pl.program_id(1)
    ns = pl.num_programs(1)
    n = n_ref[0]

    @pl.when(j == 0)
    def _():
        acc_ref[...] = jnp.zeros_like(acc_ref)

    a_s = asrc2_ref[:, 0:1]
    a_d = adst2T_ref[0:1, :]
    e = _masked_e(adj_ref[...], a_s, a_d, i, j, n)
    acc_ref[...] += jnp.dot(ht2_ref[...], e,
                            preferred_element_type=jnp.float32,
                            precision=_PREC)

    @pl.when(j == ns - 1)
    def _():
        acc = acc_ref[...]
        norm = acc[0:_NCLASS, :] / acc[_NCLASS:_NCLASS + 1, :]
        h2_ref[...] = norm.T + b2_ref[0:1, :]


@jax.jit
def kernel(adj, x, W1, att_src1, att_dst1, b1, W2, att_src2, att_dst2, b2):
    n = adj.shape[0]
    nt = pl.cdiv(n, _T)
    npad = nt * _T
    heads = _HEADS

    att1s = att_src1[0]
    att1d = att_dst1[0]
    eye = jnp.eye(heads, dtype=jnp.float32)
    asrc_m = (att1s[:, :, None] * eye[:, None, :]).reshape(heads * _NHID,
                                                           heads)
    adst_m = (att1d[:, :, None] * eye[:, None, :]).reshape(heads * _NHID,
                                                           heads)
    as2m = jnp.tile(att_src2[0, 0][:, None], (1, 8))
    ad2m = jnp.tile(att_dst2[0, 0][:, None], (1, 8))
    b1_2d = b1.reshape(1, -1)
    b2_2d = b2.reshape(1, -1)
    n_arr = jnp.full((1,), n, dtype=jnp.int32)

    nfeat = x.shape[1]
    hc = heads * _NHID
    nspec = pl.BlockSpec(memory_space=pltpu.SMEM)

    ht1, asrc1, adst1T = pl.pallas_call(
        _prep_kernel,
        grid=(nt,),
        in_specs=[
            nspec,
            pl.BlockSpec((_T, nfeat), lambda k: (k, 0)),
            pl.BlockSpec((nfeat, hc), lambda k: (0, 0)),
            pl.BlockSpec((hc, heads), lambda k: (0, 0)),
            pl.BlockSpec((hc, heads), lambda k: (0, 0)),
        ],
        out_specs=[
            pl.BlockSpec((heads, _HP, _T), lambda k: (0, 0, k)),
            pl.BlockSpec((_T, heads), lambda k: (k, 0)),
            pl.BlockSpec((heads, _T), lambda k: (0, k)),
        ],
        out_shape=[
            jax.ShapeDtypeStruct((heads, _HP, npad), jnp.float32),
            jax.ShapeDtypeStruct((npad, heads), jnp.float32),
            jax.ShapeDtypeStruct((heads, npad), jnp.float32),
        ],
    )(n_arr, x, W1, asrc_m, adst_m)

    h1, ht2, asrc2, adst2T = pl.pallas_call(
        _layer1_kernel,
        grid=(nt, nt),
        in_specs=[
            nspec,
            pl.BlockSpec((_T, _T), lambda i, j: (j, i)),
            pl.BlockSpec((heads, _HP, _T), lambda i, j: (0, 0, j)),
            pl.BlockSpec((_T, heads), lambda i, j: (j, 0)),
            pl.BlockSpec((heads, _T), lambda i, j: (0, i)),
            pl.BlockSpec((hc, _NCLASS), lambda i, j: (0, 0)),
            pl.BlockSpec((1, hc), lambda i, j: (0, 0)),
            pl.BlockSpec((_NCLASS, 8), lambda i, j: (0, 0)),
            pl.BlockSpec((_NCLASS, 8), lambda i, j: (0, 0)),
        ],
        out_specs=[
            pl.BlockSpec((_T, hc), lambda i, j: (i, 0)),
            pl.BlockSpec((_HP, _T), lambda i, j: (0, i)),
            pl.BlockSpec((_T, 8), lambda i, j: (i, 0)),
            pl.BlockSpec((8, _T), lambda i, j: (0, i)),
        ],
        out_shape=[
            jax.ShapeDtypeStruct((n, hc), jnp.float32),
            jax.ShapeDtypeStruct((_HP, npad), jnp.float32),
            jax.ShapeDtypeStruct((npad, 8), jnp.float32),
            jax.ShapeDtypeStruct((8, npad), jnp.float32),
        ],
        scratch_shapes=[pltpu.VMEM((heads, _HP, _T), jnp.float32)],
    )(n_arr, adj, ht1, asrc1, adst1T, W2, b1_2d, as2m, ad2m)

    h2 = pl.pallas_call(
        _layer2_kernel,
        grid=(nt, nt),
        in_specs=[
            nspec,
            pl.BlockSpec((_T, _T), lambda i, j: (j, i)),
            pl.BlockSpec((_HP, _T), lambda i, j: (0, j)),
            pl.BlockSpec((_T, 8), lambda i, j: (j, 0)),
            pl.BlockSpec((8, _T), lambda i, j: (0, i)),
            pl.BlockSpec((1, _NCLASS), lambda i, j: (0, 0)),
        ],
        out_specs=pl.BlockSpec((_T, _NCLASS), lambda i, j: (i, 0)),
        out_shape=jax.ShapeDtypeStruct((n, _NCLASS), jnp.float32),
        scratch_shapes=[pltpu.VMEM((_HP, _T), jnp.float32)],
    )(n_arr, adj, ht2, asrc2, adst2T, b2_2d)

    return (h2, h1, h2)


# speedup vs baseline: 69.9849x; 2.1530x over previous
import jax
import jax.numpy as jnp
from jax.experimental import pallas as pl
from jax.experimental.pallas import tpu as pltpu

_T = 1024
_HEADS = 8
_NHID = 16
_NCLASS = 16
_HP = 24
_SLOPE = 0.2
_PREC = jax.lax.Precision.HIGHEST
_PREC_E = jax.lax.Precision.DEFAULT


def _prep_kernel(n_ref, x_ref, w1_ref, asrc_m_ref, adst_m_ref,
                 ht_ref, asrc_ref, adstT_ref):
    k = pl.program_id(0)
    n = n_ref[0]
    xb = x_ref[...]
    xb = jnp.where(jnp.isnan(xb), jnp.zeros_like(xb), xb)
    h = jnp.dot(xb, w1_ref[...], preferred_element_type=jnp.float32,
                precision=_PREC)
    svalid = (jax.lax.broadcasted_iota(jnp.int32, (_T, 1), 0) + k * _T) < n
    asrc = jnp.dot(h, asrc_m_ref[...], preferred_element_type=jnp.float32,
                   precision=_PREC)
    asrc_ref[...] = jnp.where(svalid, asrc, 0.0)
    adst = jnp.dot(h, adst_m_ref[...], preferred_element_type=jnp.float32,
                   precision=_PREC)
    lvalid = svalid.T
    adstT_ref[...] = jnp.where(lvalid, adst.T, 0.0)
    zrow = jnp.zeros((1, _T), jnp.float32)
    for hh in range(_HEADS):
        blk = h[:, hh * _NHID:(hh + 1) * _NHID]
        ht_ref[hh, 0:_NHID, :] = jnp.where(lvalid, blk.T, 0.0)
        ht_ref[hh, _NHID:_NHID + 1, :] = jnp.ones((1, _T), jnp.float32)
        for pp in range(_NHID + 1, _HP):
            ht_ref[hh, pp:pp + 1, :] = zrow


def _masked_e(adjb, a_s, a_d, i, j, n):
    rows = jax.lax.broadcasted_iota(jnp.int32, (_T, _T), 0) + j * _T
    cols = jax.lax.broadcasted_iota(jnp.int32, (_T, _T), 1) + i * _T
    valid = (rows < n) & (cols < n)
    m = jnp.where(valid, adjb, 0.0) + (rows == cols).astype(jnp.float32)
    t = a_s + a_d
    t = jnp.maximum(t, _SLOPE * t)
    return jnp.exp(t) * m


def _layer1_kernel(n_ref, adj_ref, ht_ref, asrc_ref, adstT_ref, w2_ref,
                   b1_ref, as2m_ref, ad2m_ref,
                   h1_ref, ht2_ref, asrc2_ref, adst2T_ref, acc_ref):
    i = pl.program_id(0)
    j = pl.program_id(1)
    ns = pl.num_programs(1)
    n = n_ref[0]

    @pl.when(j == 0)
    def _():
        acc_ref[...] = jnp.zeros_like(acc_ref)

    adjb = adj_ref[...]
    for hh in range(_HEADS):
        a_s = asrc_ref[:, hh:hh + 1]
        a_d = adstT_ref[hh:hh + 1, :]
        e = _masked_e(adjb, a_s, a_d, i, j, n)
        acc_ref[hh] += jnp.dot(ht_ref[hh], e,
                               preferred_element_type=jnp.float32,
                               precision=_PREC_E)

    @pl.when(j == ns - 1)
    def _():
        acc = acc_ref[...]
        numer = acc[:, 0:_NHID, :]
        denom = acc[:, _NHID:_NHID + 1, :]
        norm = numer / denom
        parts = [norm[hh].T for hh in range(_HEADS)]
        h1 = jnp.concatenate(parts, axis=1) + b1_ref[0:1, :]
        h1_ref[...] = h1
        hr = jnp.maximum(h1, 0.0)
        h2f = jnp.dot(hr, w2_ref[...], preferred_element_type=jnp.float32,
                      precision=_PREC)
        ht2_ref[0:_NCLASS, :] = h2f.T
        ht2_ref[_NCLASS:_NCLASS + 1, :] = jnp.ones((1, _T), jnp.float32)
        for pp in range(_NCLASS + 1, _HP):
            ht2_ref[pp:pp + 1, :] = jnp.zeros((1, _T), jnp.float32)
        asrc2_ref[...] = jnp.dot(h2f, as2m_ref[...],
                                 preferred_element_type=jnp.float32,
                                 precision=_PREC)
        adst2T_ref[...] = jnp.dot(h2f, ad2m_ref[...],
                                  preferred_element_type=jnp.float32,
                                  precision=_PREC).T


def _layer2_kernel(n_ref, adj_ref, ht2_ref, asrc2_ref, adst2T_ref, b2_ref,
                   h2_ref, acc_ref):
    i = pl.program_id(0)
    j = pl.program_id(1)
    ns = pl.num_programs(1)
    n = n_ref[0]

    @pl.when(j == 0)
    def _():
        acc_ref[...] = jnp.zeros_like(acc_ref)

    a_s = asrc2_ref[:, 0:1]
    a_d = adst2T_ref[0:1, :]
    e = _masked_e(adj_ref[...], a_s, a_d, i, j, n)
    acc_ref[...] += jnp.dot(ht2_ref[...], e,
                            preferred_element_type=jnp.float32,
                            precision=_PREC_E)

    @pl.when(j == ns - 1)
    def _():
        acc = acc_ref[...]
        norm = acc[0:_NCLASS, :] / acc[_NCLASS:_NCLASS + 1, :]
        h2_ref[...] = norm.T + b2_ref[0:1, :]


@jax.jit
def kernel(adj, x, W1, att_src1, att_dst1, b1, W2, att_src2, att_dst2, b2):
    n = adj.shape[0]
    nt = pl.cdiv(n, _T)
    npad = nt * _T
    heads = _HEADS

    att1s = att_src1[0]
    att1d = att_dst1[0]
    eye = jnp.eye(heads, dtype=jnp.float32)
    asrc_m = (att1s[:, :, None] * eye[:, None, :]).reshape(heads * _NHID,
                                                           heads)
    adst_m = (att1d[:, :, None] * eye[:, None, :]).reshape(heads * _NHID,
                                                           heads)
    as2m = jnp.tile(att_src2[0, 0][:, None], (1, 8))
    ad2m = jnp.tile(att_dst2[0, 0][:, None], (1, 8))
    b1_2d = b1.reshape(1, -1)
    b2_2d = b2.reshape(1, -1)
    n_arr = jnp.full((1,), n, dtype=jnp.int32)

    nfeat = x.shape[1]
    hc = heads * _NHID
    nspec = pl.BlockSpec(memory_space=pltpu.SMEM)

    ht1, asrc1, adst1T = pl.pallas_call(
        _prep_kernel,
        grid=(nt,),
        in_specs=[
            nspec,
            pl.BlockSpec((_T, nfeat), lambda k: (k, 0)),
            pl.BlockSpec((nfeat, hc), lambda k: (0, 0)),
            pl.BlockSpec((hc, heads), lambda k: (0, 0)),
            pl.BlockSpec((hc, heads), lambda k: (0, 0)),
        ],
        out_specs=[
            pl.BlockSpec((heads, _HP, _T), lambda k: (0, 0, k)),
            pl.BlockSpec((_T, heads), lambda k: (k, 0)),
            pl.BlockSpec((heads, _T), lambda k: (0, k)),
        ],
        out_shape=[
            jax.ShapeDtypeStruct((heads, _HP, npad), jnp.float32),
            jax.ShapeDtypeStruct((npad, heads), jnp.float32),
            jax.ShapeDtypeStruct((heads, npad), jnp.float32),
        ],
    )(n_arr, x, W1, asrc_m, adst_m)

    h1, ht2, asrc2, adst2T = pl.pallas_call(
        _layer1_kernel,
        grid=(nt, nt),
        in_specs=[
            nspec,
            pl.BlockSpec((_T, _T), lambda i, j: (j, i)),
            pl.BlockSpec((heads, _HP, _T), lambda i, j: (0, 0, j)),
            pl.BlockSpec((_T, heads), lambda i, j: (j, 0)),
            pl.BlockSpec((heads, _T), lambda i, j: (0, i)),
            pl.BlockSpec((hc, _NCLASS), lambda i, j: (0, 0)),
            pl.BlockSpec((1, hc), lambda i, j: (0, 0)),
            pl.BlockSpec((_NCLASS, 8), lambda i, j: (0, 0)),
            pl.BlockSpec((_NCLASS, 8), lambda i, j: (0, 0)),
        ],
        out_specs=[
            pl.BlockSpec((_T, hc), lambda i, j: (i, 0)),
            pl.BlockSpec((_HP, _T), lambda i, j: (0, i)),
            pl.BlockSpec((_T, 8), lambda i, j: (i, 0)),
            pl.BlockSpec((8, _T), lambda i, j: (0, i)),
        ],
        out_shape=[
            jax.ShapeDtypeStruct((n, hc), jnp.float32),
            jax.ShapeDtypeStruct((_HP, npad), jnp.float32),
            jax.ShapeDtypeStruct((npad, 8), jnp.float32),
            jax.ShapeDtypeStruct((8, npad), jnp.float32),
        ],
        scratch_shapes=[pltpu.VMEM((heads, _HP, _T), jnp.float32)],
    )(n_arr, adj, ht1, asrc1, adst1T, W2, b1_2d, as2m, ad2m)

    h2 = pl.pallas_call(
        _layer2_kernel,
        grid=(nt, nt),
        in_specs=[
            nspec,
            pl.BlockSpec((_T, _T), lambda i, j: (j, i)),
            pl.BlockSpec((_HP, _T), lambda i, j: (0, j)),
            pl.BlockSpec((_T, 8), lambda i, j: (j, 0)),
            pl.BlockSpec((8, _T), lambda i, j: (0, i)),
            pl.BlockSpec((1, _NCLASS), lambda i, j: (0, 0)),
        ],
        out_specs=pl.BlockSpec((_T, _NCLASS), lambda i, j: (i, 0)),
        out_shape=jax.ShapeDtypeStruct((n, _NCLASS), jnp.float32),
        scratch_shapes=[pltpu.VMEM((_HP, _T), jnp.float32)],
    )(n_arr, adj, ht2, asrc2, adst2T, b2_2d)

    return (h2, h1, h2)


# speedup vs baseline: 88.7121x; 1.2676x over previous
import jax
import jax.numpy as jnp
from jax.experimental import pallas as pl
from jax.experimental.pallas import tpu as pltpu

_T = 1024
_HEADS = 8
_NHID = 16
_NCLASS = 16
_HP = 24
_SLOPE = 0.2
_PREC = jax.lax.Precision.HIGHEST
_PREC_E = jax.lax.Precision.DEFAULT


def _loop_weight(ps, qs, pdT, qdT):
    return jnp.maximum(ps.T * pdT, qs.T * qdT)


def _prep_kernel(n_ref, x_ref, w1_ref, asrc_m_ref, adst_m_ref,
                 ht_ref, ps_ref, qs_ref, pdT_ref, qdT_ref, wT_ref):
    k = pl.program_id(0)
    n = n_ref[0]
    xb = x_ref[...]
    xb = jnp.where(jnp.isnan(xb), jnp.zeros_like(xb), xb)
    h = jnp.dot(xb, w1_ref[...], preferred_element_type=jnp.float32,
                precision=_PREC)
    svalid = (jax.lax.broadcasted_iota(jnp.int32, (_T, 1), 0) + k * _T) < n
    asrc = jnp.dot(h, asrc_m_ref[...], preferred_element_type=jnp.float32,
                   precision=_PREC)
    asrc = jnp.where(svalid, asrc, 0.0)
    adst = jnp.dot(h, adst_m_ref[...], preferred_element_type=jnp.float32,
                   precision=_PREC)
    adst = jnp.where(svalid, adst, 0.0)
    ps = jnp.exp(asrc)
    qs = jnp.exp(_SLOPE * asrc)
    pdT = jnp.exp(adst).T
    qdT = jnp.exp(_SLOPE * adst).T
    ps_ref[...] = ps
    qs_ref[...] = qs
    pdT_ref[...] = pdT
    qdT_ref[...] = qdT
    wT_ref[...] = _loop_weight(ps, qs, pdT, qdT)
    lvalid = svalid.T
    for hh in range(_HEADS):
        blk = h[:, hh * _NHID:(hh + 1) * _NHID]
        ht_ref[hh, 0:_NHID, :] = jnp.where(lvalid, blk.T, 0.0)
        ht_ref[hh, _NHID:_NHID + 1, :] = jnp.ones((1, _T), jnp.float32)
        for pp in range(_NHID + 1, _HP):
            ht_ref[hh, pp:pp + 1, :] = jnp.zeros((1, _T), jnp.float32)


def _masked_adj(adjb, i, j, n):
    rows = jax.lax.broadcasted_iota(jnp.int32, (_T, _T), 0) + j * _T
    cols = jax.lax.broadcasted_iota(jnp.int32, (_T, _T), 1) + i * _T
    return jnp.where((rows < n) & (cols < n), adjb, 0.0)


def _layer1_kernel(n_ref, adj_ref, ht_ref, ps_ref, qs_ref, pdT_ref,
                   qdT_ref, wT_ref, htd_ref, w2_ref, b1_ref, as2m_ref,
                   ad2m_ref,
                   h1_ref, ht2_ref, ps2_ref, qs2_ref, pd2T_ref, qd2T_ref,
                   w2T_ref, acc_ref):
    i = pl.program_id(0)
    j = pl.program_id(1)
    ns = pl.num_programs(1)
    n = n_ref[0]

    @pl.when(j == 0)
    def _():
        acc_ref[...] = jnp.zeros_like(acc_ref)

    madj = _masked_adj(adj_ref[...], i, j, n)
    for hh in range(_HEADS):
        e = jnp.maximum(ps_ref[:, hh:hh + 1] * pdT_ref[hh:hh + 1, :],
                        qs_ref[:, hh:hh + 1] * qdT_ref[hh:hh + 1, :])
        e = e * madj
        acc_ref[hh] += jnp.dot(ht_ref[hh], e,
                               preferred_element_type=jnp.float32,
                               precision=_PREC_E)

    @pl.when(j == ns - 1)
    def _():
        acc = acc_ref[...]
        wT = wT_ref[...]
        acc = acc + wT[:, None, :] * htd_ref[...]
        numer = acc[:, 0:_NHID, :]
        denom = acc[:, _NHID:_NHID + 1, :]
        norm = numer / denom
        parts = [norm[hh].T for hh in range(_HEADS)]
        h1 = jnp.concatenate(parts, axis=1) + b1_ref[0:1, :]
        h1_ref[...] = h1
        hr = jnp.maximum(h1, 0.0)
        h2f = jnp.dot(hr, w2_ref[...], preferred_element_type=jnp.float32,
                      precision=_PREC)
        ht2_ref[0:_NCLASS, :] = h2f.T
        ht2_ref[_NCLASS:_NCLASS + 1, :] = jnp.ones((1, _T), jnp.float32)
        for pp in range(_NCLASS + 1, _HP):
            ht2_ref[pp:pp + 1, :] = jnp.zeros((1, _T), jnp.float32)
        a2s = jnp.dot(h2f, as2m_ref[...], preferred_element_type=jnp.float32,
                      precision=_PREC)
        a2d = jnp.dot(h2f, ad2m_ref[...], preferred_element_type=jnp.float32,
                      precision=_PREC)
        ps2 = jnp.exp(a2s)
        qs2 = jnp.exp(_SLOPE * a2s)
        pd2T = jnp.exp(a2d).T
        qd2T = jnp.exp(_SLOPE * a2d).T
        ps2_ref[...] = ps2
        qs2_ref[...] = qs2
        pd2T_ref[...] = pd2T
        qd2T_ref[...] = qd2T
        w2T_ref[...] = _loop_weight(ps2, qs2, pd2T, qd2T)


def _layer2_kernel(n_ref, adj_ref, ht2_ref, ps2_ref, qs2_ref, pd2T_ref,
                   qd2T_ref, w2T_ref, ht2d_ref, b2_ref,
                   h2_ref, acc_ref):
    i = pl.program_id(0)
    j = pl.program_id(1)
    ns = pl.num_programs(1)
    n = n_ref[0]

    @pl.when(j == 0)
    def _():
        acc_ref[...] = jnp.zeros_like(acc_ref)

    madj = _masked_adj(adj_ref[...], i, j, n)
    e = jnp.maximum(ps2_ref[:, 0:1] * pd2T_ref[0:1, :],
                    qs2_ref[:, 0:1] * qd2T_ref[0:1, :])
    e = e * madj
    acc_ref[...] += jnp.dot(ht2_ref[...], e,
                            preferred_element_type=jnp.float32,
                            precision=_PREC_E)

    @pl.when(j == ns - 1)
    def _():
        acc = acc_ref[...] + w2T_ref[0:1, :] * ht2d_ref[...]
        norm = acc[0:_NCLASS, :] / acc[_NCLASS:_NCLASS + 1, :]
        h2_ref[...] = norm.T + b2_ref[0:1, :]


@jax.jit
def kernel(adj, x, W1, att_src1, att_dst1, b1, W2, att_src2, att_dst2, b2):
    n = adj.shape[0]
    nt = pl.cdiv(n, _T)
    npad = nt * _T
    heads = _HEADS

    att1s = att_src1[0]
    att1d = att_dst1[0]
    eye = jnp.eye(heads, dtype=jnp.float32)
    asrc_m = (att1s[:, :, None] * eye[:, None, :]).reshape(heads * _NHID,
                                                           heads)
    adst_m = (att1d[:, :, None] * eye[:, None, :]).reshape(heads * _NHID,
                                                           heads)
    as2m = jnp.tile(att_src2[0, 0][:, None], (1, 8))
    ad2m = jnp.tile(att_dst2[0, 0][:, None], (1, 8))
    b1_2d = b1.reshape(1, -1)
    b2_2d = b2.reshape(1, -1)
    n_arr = jnp.full((1,), n, dtype=jnp.int32)

    nfeat = x.shape[1]
    hc = heads * _NHID
    nspec = pl.BlockSpec(memory_space=pltpu.SMEM)

    def nvec(idx):
        return pl.BlockSpec((_T, 8), idx), jax.ShapeDtypeStruct(
            (npad, 8), jnp.float32)

    def lvec(idx):
        return pl.BlockSpec((8, _T), idx), jax.ShapeDtypeStruct(
            (8, npad), jnp.float32)

    p_specs = [
        nvec(lambda k: (k, 0)), nvec(lambda k: (k, 0)),
        lvec(lambda k: (0, k)), lvec(lambda k: (0, k)),
        lvec(lambda k: (0, k)),
    ]
    ht1, ps1, qs1, pd1T, qd1T, w1T = pl.pallas_call(
        _prep_kernel,
        grid=(nt,),
        in_specs=[
            nspec,
            pl.BlockSpec((_T, nfeat), lambda k: (k, 0)),
            pl.BlockSpec((nfeat, hc), lambda k: (0, 0)),
            pl.BlockSpec((hc, heads), lambda k: (0, 0)),
            pl.BlockSpec((hc, heads), lambda k: (0, 0)),
        ],
        out_specs=[pl.BlockSpec((heads, _HP, _T), lambda k: (0, 0, k))] +
                  [s for s, _ in p_specs],
        out_shape=[jax.ShapeDtypeStruct((heads, _HP, npad), jnp.float32)] +
                  [sh for _, sh in p_specs],
    )(n_arr, x, W1, asrc_m, adst_m)

    q_specs = [
        nvec(lambda i, j: (j, 0)), nvec(lambda i, j: (j, 0)),
        lvec(lambda i, j: (0, i)), lvec(lambda i, j: (0, i)),
        lvec(lambda i, j: (0, i)),
    ]
    h1, ht2, ps2, qs2, pd2T, qd2T, w2T = pl.pallas_call(
        _layer1_kernel,
        grid=(nt, nt),
        in_specs=[
            nspec,
            pl.BlockSpec((_T, _T), lambda i, j: (j, i)),
            pl.BlockSpec((heads, _HP, _T), lambda i, j: (0, 0, j)),
        ] + [s for s, _ in q_specs] + [
            pl.BlockSpec((heads, _HP, _T), lambda i, j: (0, 0, i)),
            pl.BlockSpec((hc, _NCLASS), lambda i, j: (0, 0)),
            pl.BlockSpec((1, hc), lambda i, j: (0, 0)),
            pl.BlockSpec((_NCLASS, 8), lambda i, j: (0, 0)),
            pl.BlockSpec((_NCLASS, 8), lambda i, j: (0, 0)),
        ],
        out_specs=[
            pl.BlockSpec((_T, hc), lambda i, j: (i, 0)),
            pl.BlockSpec((_HP, _T), lambda i, j: (0, i)),
            pl.BlockSpec((_T, 8), lambda i, j: (i, 0)),
            pl.BlockSpec((_T, 8), lambda i, j: (i, 0)),
            pl.BlockSpec((8, _T), lambda i, j: (0, i)),
            pl.BlockSpec((8, _T), lambda i, j: (0, i)),
            pl.BlockSpec((8, _T), lambda i, j: (0, i)),
        ],
        out_shape=[
            jax.ShapeDtypeStruct((n, hc), jnp.float32),
            jax.ShapeDtypeStruct((_HP, npad), jnp.float32),
            jax.ShapeDtypeStruct((npad, 8), jnp.float32),
            jax.ShapeDtypeStruct((npad, 8), jnp.float32),
            jax.ShapeDtypeStruct((8, npad), jnp.float32),
            jax.ShapeDtypeStruct((8, npad), jnp.float32),
            jax.ShapeDtypeStruct((8, npad), jnp.float32),
        ],
        scratch_shapes=[pltpu.VMEM((heads, _HP, _T), jnp.float32)],
    )(n_arr, adj, ht1, ps1, qs1, pd1T, qd1T, w1T, ht1, W2, b1_2d,
      as2m, ad2m)

    h2 = pl.pallas_call(
        _layer2_kernel,
        grid=(nt, nt),
        in_specs=[
            nspec,
            pl.BlockSpec((_T, _T), lambda i, j: (j, i)),
            pl.BlockSpec((_HP, _T), lambda i, j: (0, j)),
            pl.BlockSpec((_T, 8), lambda i, j: (j, 0)),
            pl.BlockSpec((_T, 8), lambda i, j: (j, 0)),
            pl.BlockSpec((8, _T), lambda i, j: (0, i)),
            pl.BlockSpec((8, _T), lambda i, j: (0, i)),
            pl.BlockSpec((8, _T), lambda i, j: (0, i)),
            pl.BlockSpec((_HP, _T), lambda i, j: (0, i)),
            pl.BlockSpec((1, _NCLASS), lambda i, j: (0, 0)),
        ],
        out_specs=pl.BlockSpec((_T, _NCLASS), lambda i, j: (i, 0)),
        out_shape=jax.ShapeDtypeStruct((n, _NCLASS), jnp.float32),
        scratch_shapes=[pltpu.VMEM((_HP, _T), jnp.float32)],
    )(n_arr, adj, ht2, ps2, qs2, pd2T, qd2T, w2T, ht2, b2_2d)

    return (h2, h1, h2)


# speedup vs baseline: 99.8703x; 1.1258x over previous
import jax
import jax.numpy as jnp
from jax.experimental import pallas as pl
from jax.experimental.pallas import tpu as pltpu

_T = 1024
_HEADS = 8
_NHID = 16
_NCLASS = 16
_HP = 24
_SLOPE = 0.2
_PREC = jax.lax.Precision.HIGHEST
_PREC_E = jax.lax.Precision.DEFAULT


def _loop_weight(ps, qs, pdT, qdT):
    return jnp.maximum(ps.T * pdT, qs.T * qdT)


def _prep_kernel(n_ref, x_ref, w1_ref, asrc_m_ref, adst_m_ref,
                 ht_ref, ps_ref, qs_ref, pdT_ref, qdT_ref, wT_ref):
    k = pl.program_id(0)
    n = n_ref[0]
    xb = x_ref[...]
    xb = jnp.where(jnp.isnan(xb), jnp.zeros_like(xb), xb)
    h = jnp.dot(xb, w1_ref[...], preferred_element_type=jnp.float32,
                precision=_PREC)
    svalid = (jax.lax.broadcasted_iota(jnp.int32, (_T, 1), 0) + k * _T) < n
    asrc = jnp.dot(h, asrc_m_ref[...], preferred_element_type=jnp.float32,
                   precision=_PREC)
    asrc = jnp.where(svalid, asrc, 0.0)
    adst = jnp.dot(h, adst_m_ref[...], preferred_element_type=jnp.float32,
                   precision=_PREC)
    adst = jnp.where(svalid, adst, 0.0)
    ps = jnp.exp(asrc)
    qs = jnp.exp(_SLOPE * asrc)
    pdT = jnp.exp(adst).T
    qdT = jnp.exp(_SLOPE * adst).T
    ps_ref[...] = ps.astype(jnp.bfloat16)
    qs_ref[...] = qs.astype(jnp.bfloat16)
    pdT_ref[...] = pdT.astype(jnp.bfloat16)
    qdT_ref[...] = qdT.astype(jnp.bfloat16)
    wT_ref[...] = _loop_weight(ps, qs, pdT, qdT)
    lvalid = svalid.T
    for hh in range(_HEADS):
        blk = h[:, hh * _NHID:(hh + 1) * _NHID]
        ht_ref[hh, 0:_NHID, :] = jnp.where(lvalid, blk.T,
                                           0.0).astype(jnp.bfloat16)
        ht_ref[hh, _NHID:_NHID + 1, :] = jnp.ones((1, _T), jnp.bfloat16)
        for pp in range(_NHID + 1, _HP):
            ht_ref[hh, pp:pp + 1, :] = jnp.zeros((1, _T), jnp.bfloat16)


def _masked_adj(adjb, i, j, n):
    rowv = (jax.lax.broadcasted_iota(jnp.int32, (_T, 1), 0) + j * _T) < n
    colv = (jax.lax.broadcasted_iota(jnp.int32, (1, _T), 1) + i * _T) < n
    return jnp.where(rowv & colv, adjb, 0.0).astype(jnp.bfloat16)


def _layer1_kernel(n_ref, adj_ref, ht_ref, ps_ref, qs_ref, pdT_ref,
                   qdT_ref, wT_ref, htd_ref, w2_ref, b1_ref, as2m_ref,
                   ad2m_ref,
                   h1_ref, ht2_ref, ps2_ref, qs2_ref, pd2T_ref, qd2T_ref,
                   w2T_ref, acc_ref):
    i = pl.program_id(0)
    j = pl.program_id(1)
    ns = pl.num_programs(1)
    n = n_ref[0]

    @pl.when(j == 0)
    def _():
        acc_ref[...] = jnp.zeros_like(acc_ref)

    madj = _masked_adj(adj_ref[...], i, j, n)
    for hh in range(_HEADS):
        e = jnp.maximum(ps_ref[:, hh:hh + 1] * pdT_ref[hh:hh + 1, :],
                        qs_ref[:, hh:hh + 1] * qdT_ref[hh:hh + 1, :])
        e = e * madj
        acc_ref[hh] += jnp.dot(ht_ref[hh], e,
                               preferred_element_type=jnp.float32,
                               precision=_PREC_E)

    @pl.when(j == ns - 1)
    def _():
        acc = acc_ref[...]
        wT = wT_ref[...]
        acc = acc + wT[:, None, :] * htd_ref[...].astype(jnp.float32)
        numer = acc[:, 0:_NHID, :]
        denom = acc[:, _NHID:_NHID + 1, :]
        norm = numer / denom
        parts = [norm[hh].T for hh in range(_HEADS)]
        h1 = jnp.concatenate(parts, axis=1) + b1_ref[0:1, :]
        h1_ref[...] = h1
        hr = jnp.maximum(h1, 0.0)
        h2f = jnp.dot(hr, w2_ref[...], preferred_element_type=jnp.float32,
                      precision=_PREC)
        ht2_ref[0:_NCLASS, :] = h2f.T.astype(jnp.bfloat16)
        ht2_ref[_NCLASS:_NCLASS + 1, :] = jnp.ones((1, _T), jnp.bfloat16)
        for pp in range(_NCLASS + 1, _HP):
            ht2_ref[pp:pp + 1, :] = jnp.zeros((1, _T), jnp.bfloat16)
        a2s = jnp.dot(h2f, as2m_ref[...], preferred_element_type=jnp.float32,
                      precision=_PREC)
        a2d = jnp.dot(h2f, ad2m_ref[...], preferred_element_type=jnp.float32,
                      precision=_PREC)
        ps2 = jnp.exp(a2s)
        qs2 = jnp.exp(_SLOPE * a2s)
        pd2T = jnp.exp(a2d).T
        qd2T = jnp.exp(_SLOPE * a2d).T
        ps2_ref[...] = ps2.astype(jnp.bfloat16)
        qs2_ref[...] = qs2.astype(jnp.bfloat16)
        pd2T_ref[...] = pd2T.astype(jnp.bfloat16)
        qd2T_ref[...] = qd2T.astype(jnp.bfloat16)
        w2T_ref[...] = _loop_weight(ps2, qs2, pd2T, qd2T)


def _layer2_kernel(n_ref, adj_ref, ht2_ref, ps2_ref, qs2_ref, pd2T_ref,
                   qd2T_ref, w2T_ref, ht2d_ref, b2_ref,
                   h2_ref, acc_ref):
    i = pl.program_id(0)
    j = pl.program_id(1)
    ns = pl.num_programs(1)
    n = n_ref[0]

    @pl.when(j == 0)
    def _():
        acc_ref[...] = jnp.zeros_like(acc_ref)

    madj = _masked_adj(adj_ref[...], i, j, n)
    e = jnp.maximum(ps2_ref[:, 0:1] * pd2T_ref[0:1, :],
                    qs2_ref[:, 0:1] * qd2T_ref[0:1, :])
    e = e * madj
    acc_ref[...] += jnp.dot(ht2_ref[...], e,
                            preferred_element_type=jnp.float32,
                            precision=_PREC_E)

    @pl.when(j == ns - 1)
    def _():
        acc = acc_ref[...] + w2T_ref[0:1, :] * ht2d_ref[...].astype(jnp.float32)
        norm = acc[0:_NCLASS, :] / acc[_NCLASS:_NCLASS + 1, :]
        h2_ref[...] = norm.T + b2_ref[0:1, :]


@jax.jit
def kernel(adj, x, W1, att_src1, att_dst1, b1, W2, att_src2, att_dst2, b2):
    n = adj.shape[0]
    nt = pl.cdiv(n, _T)
    npad = nt * _T
    heads = _HEADS

    att1s = att_src1[0]
    att1d = att_dst1[0]
    eye = jnp.eye(heads, dtype=jnp.float32)
    asrc_m = (att1s[:, :, None] * eye[:, None, :]).reshape(heads * _NHID,
                                                           heads)
    adst_m = (att1d[:, :, None] * eye[:, None, :]).reshape(heads * _NHID,
                                                           heads)
    as2m = jnp.tile(att_src2[0, 0][:, None], (1, 8))
    ad2m = jnp.tile(att_dst2[0, 0][:, None], (1, 8))
    b1_2d = b1.reshape(1, -1)
    b2_2d = b2.reshape(1, -1)
    n_arr = jnp.full((1,), n, dtype=jnp.int32)

    nfeat = x.shape[1]
    hc = heads * _NHID
    nspec = pl.BlockSpec(memory_space=pltpu.SMEM)

    def nvec(idx, dt=jnp.bfloat16):
        return pl.BlockSpec((_T, 8), idx), jax.ShapeDtypeStruct(
            (npad, 8), dt)

    def lvec(idx, dt=jnp.bfloat16):
        return pl.BlockSpec((8, _T), idx), jax.ShapeDtypeStruct(
            (8, npad), dt)

    p_specs = [
        nvec(lambda k: (k, 0)), nvec(lambda k: (k, 0)),
        lvec(lambda k: (0, k)), lvec(lambda k: (0, k)),
        lvec(lambda k: (0, k), jnp.float32),
    ]
    ht1, ps1, qs1, pd1T, qd1T, w1T = pl.pallas_call(
        _prep_kernel,
        grid=(nt,),
        in_specs=[
            nspec,
            pl.BlockSpec((_T, nfeat), lambda k: (k, 0)),
            pl.BlockSpec((nfeat, hc), lambda k: (0, 0)),
            pl.BlockSpec((hc, heads), lambda k: (0, 0)),
            pl.BlockSpec((hc, heads), lambda k: (0, 0)),
        ],
        out_specs=[pl.BlockSpec((heads, _HP, _T), lambda k: (0, 0, k))] +
                  [s for s, _ in p_specs],
        out_shape=[jax.ShapeDtypeStruct((heads, _HP, npad), jnp.bfloat16)] +
                  [sh for _, sh in p_specs],
    )(n_arr, x, W1, asrc_m, adst_m)

    q_specs = [
        nvec(lambda i, j: (j, 0)), nvec(lambda i, j: (j, 0)),
        lvec(lambda i, j: (0, i)), lvec(lambda i, j: (0, i)),
        lvec(lambda i, j: (0, i), jnp.float32),
    ]
    h1, ht2, ps2, qs2, pd2T, qd2T, w2T = pl.pallas_call(
        _layer1_kernel,
        grid=(nt, nt),
        in_specs=[
            nspec,
            pl.BlockSpec((_T, _T), lambda i, j: (j, i)),
            pl.BlockSpec((heads, _HP, _T), lambda i, j: (0, 0, j)),
        ] + [s for s, _ in q_specs] + [
            pl.BlockSpec((heads, _HP, _T), lambda i, j: (0, 0, i)),
            pl.BlockSpec((hc, _NCLASS), lambda i, j: (0, 0)),
            pl.BlockSpec((1, hc), lambda i, j: (0, 0)),
            pl.BlockSpec((_NCLASS, 8), lambda i, j: (0, 0)),
            pl.BlockSpec((_NCLASS, 8), lambda i, j: (0, 0)),
        ],
        out_specs=[
            pl.BlockSpec((_T, hc), lambda i, j: (i, 0)),
            pl.BlockSpec((_HP, _T), lambda i, j: (0, i)),
            pl.BlockSpec((_T, 8), lambda i, j: (i, 0)),
            pl.BlockSpec((_T, 8), lambda i, j: (i, 0)),
            pl.BlockSpec((8, _T), lambda i, j: (0, i)),
            pl.BlockSpec((8, _T), lambda i, j: (0, i)),
            pl.BlockSpec((8, _T), lambda i, j: (0, i)),
        ],
        out_shape=[
            jax.ShapeDtypeStruct((n, hc), jnp.float32),
            jax.ShapeDtypeStruct((_HP, npad), jnp.bfloat16),
            jax.ShapeDtypeStruct((npad, 8), jnp.bfloat16),
            jax.ShapeDtypeStruct((npad, 8), jnp.bfloat16),
            jax.ShapeDtypeStruct((8, npad), jnp.bfloat16),
            jax.ShapeDtypeStruct((8, npad), jnp.bfloat16),
            jax.ShapeDtypeStruct((8, npad), jnp.float32),
        ],
        scratch_shapes=[pltpu.VMEM((heads, _HP, _T), jnp.float32)],
    )(n_arr, adj, ht1, ps1, qs1, pd1T, qd1T, w1T, ht1, W2, b1_2d,
      as2m, ad2m)

    h2 = pl.pallas_call(
        _layer2_kernel,
        grid=(nt, nt),
        in_specs=[
            nspec,
            pl.BlockSpec((_T, _T), lambda i, j: (j, i)),
            pl.BlockSpec((_HP, _T), lambda i, j: (0, j)),
            pl.BlockSpec((_T, 8), lambda i, j: (j, 0)),
            pl.BlockSpec((_T, 8), lambda i, j: (j, 0)),
            pl.BlockSpec((8, _T), lambda i, j: (0, i)),
            pl.BlockSpec((8, _T), lambda i, j: (0, i)),
            pl.BlockSpec((8, _T), lambda i, j: (0, i)),
            pl.BlockSpec((_HP, _T), lambda i, j: (0, i)),
            pl.BlockSpec((1, _NCLASS), lambda i, j: (0, 0)),
        ],
        out_specs=pl.BlockSpec((_T, _NCLASS), lambda i, j: (i, 0)),
        out_shape=jax.ShapeDtypeStruct((n, _NCLASS), jnp.float32),
        scratch_shapes=[pltpu.VMEM((_HP, _T), jnp.float32)],
    )(n_arr, adj, ht2, ps2, qs2, pd2T, qd2T, w2T, ht2, b2_2d)

    return (h2, h1, h2)


# speedup vs baseline: 105.1682x; 1.0530x over previous
import jax
import jax.numpy as jnp
from jax.experimental import pallas as pl
from jax.experimental.pallas import tpu as pltpu

_T = 1024
_TSM = 2
_HEADS = 8
_NHID = 16
_NCLASS = 16
_HP = 24
_SLOPE = 0.2
_PREC = jax.lax.Precision.HIGHEST
_PREC_E = jax.lax.Precision.DEFAULT


def _loop_weight(ps, qs, pdT, qdT):
    return jnp.maximum(ps.T * pdT, qs.T * qdT)


def _prep_kernel(n_ref, x_ref, w1_ref, asrc_m_ref, adst_m_ref,
                 ht_ref, ps_ref, qs_ref, pdT_ref, qdT_ref, wT_ref):
    k = pl.program_id(0)
    n = n_ref[0]
    xb = x_ref[...]
    xb = jnp.where(jnp.isnan(xb), jnp.zeros_like(xb), xb)
    h = jnp.dot(xb, w1_ref[...], preferred_element_type=jnp.float32,
                precision=_PREC)
    svalid = (jax.lax.broadcasted_iota(jnp.int32, (_T, 1), 0) + k * _T) < n
    asrc = jnp.dot(h, asrc_m_ref[...], preferred_element_type=jnp.float32,
                   precision=_PREC)
    asrc = jnp.where(svalid, asrc, 0.0)
    adst = jnp.dot(h, adst_m_ref[...], preferred_element_type=jnp.float32,
                   precision=_PREC)
    adst = jnp.where(svalid, adst, 0.0)
    ps = jnp.exp(asrc)
    qs = jnp.exp(_SLOPE * asrc)
    pdT = jnp.exp(adst).T
    qdT = jnp.exp(_SLOPE * adst).T
    ps_ref[...] = ps.astype(jnp.bfloat16)
    qs_ref[...] = qs.astype(jnp.bfloat16)
    pdT_ref[...] = pdT.astype(jnp.bfloat16)
    qdT_ref[...] = qdT.astype(jnp.bfloat16)
    wT_ref[...] = _loop_weight(ps, qs, pdT, qdT)
    lvalid = svalid.T
    for hh in range(_HEADS):
        blk = h[:, hh * _NHID:(hh + 1) * _NHID]
        ht_ref[hh, 0:_NHID, :] = jnp.where(lvalid, blk.T,
                                           0.0).astype(jnp.bfloat16)
        ht_ref[hh, _NHID:_NHID + 1, :] = jnp.ones((1, _T), jnp.bfloat16)
        for pp in range(_NHID + 1, _HP):
            ht_ref[hh, pp:pp + 1, :] = jnp.zeros((1, _T), jnp.bfloat16)


def _masked_adj(adjb, i, j, n):
    ts = _TSM * _T
    rowv = (jax.lax.broadcasted_iota(jnp.int32, (ts, 1), 0) + j * ts) < n
    colv = (jax.lax.broadcasted_iota(jnp.int32, (1, _T), 1) + i * _T) < n
    return jnp.where(rowv & colv, adjb, 0.0).astype(jnp.bfloat16)


def _layer1_kernel(n_ref, adj_ref, ht_ref, ps_ref, qs_ref, pdT_ref,
                   qdT_ref, wT_ref, htd_ref, w2_ref, b1_ref, as2m_ref,
                   ad2m_ref,
                   h1_ref, ht2_ref, ps2_ref, qs2_ref, pd2T_ref, qd2T_ref,
                   w2T_ref, acc_ref):
    i = pl.program_id(0)
    j = pl.program_id(1)
    ns = pl.num_programs(1)
    n = n_ref[0]

    @pl.when(j == 0)
    def _():
        acc_ref[...] = jnp.zeros_like(acc_ref)

    madj = _masked_adj(adj_ref[...], i, j, n)
    for hh in range(_HEADS):
        e = jnp.maximum(ps_ref[:, hh:hh + 1] * pdT_ref[hh:hh + 1, :],
                        qs_ref[:, hh:hh + 1] * qdT_ref[hh:hh + 1, :])
        e = e * madj
        acc_ref[hh] += jnp.dot(ht_ref[hh], e,
                               preferred_element_type=jnp.float32,
                               precision=_PREC_E)

    @pl.when(j == ns - 1)
    def _():
        acc = acc_ref[...]
        wT = wT_ref[...]
        acc = acc + wT[:, None, :] * htd_ref[...].astype(jnp.float32)
        numer = acc[:, 0:_NHID, :]
        denom = acc[:, _NHID:_NHID + 1, :]
        norm = numer / denom
        parts = [norm[hh].T for hh in range(_HEADS)]
        h1 = jnp.concatenate(parts, axis=1) + b1_ref[0:1, :]
        h1_ref[...] = h1
        hr = jnp.maximum(h1, 0.0)
        h2f = jnp.dot(hr, w2_ref[...], preferred_element_type=jnp.float32,
                      precision=_PREC)
        ht2_ref[0:_NCLASS, :] = h2f.T.astype(jnp.bfloat16)
        ht2_ref[_NCLASS:_NCLASS + 1, :] = jnp.ones((1, _T), jnp.bfloat16)
        for pp in range(_NCLASS + 1, _HP):
            ht2_ref[pp:pp + 1, :] = jnp.zeros((1, _T), jnp.bfloat16)
        a2s = jnp.dot(h2f, as2m_ref[...], preferred_element_type=jnp.float32,
                      precision=_PREC)
        a2d = jnp.dot(h2f, ad2m_ref[...], preferred_element_type=jnp.float32,
                      precision=_PREC)
        ps2 = jnp.exp(a2s)
        qs2 = jnp.exp(_SLOPE * a2s)
        pd2T = jnp.exp(a2d).T
        qd2T = jnp.exp(_SLOPE * a2d).T
        ps2_ref[...] = ps2.astype(jnp.bfloat16)
        qs2_ref[...] = qs2.astype(jnp.bfloat16)
        pd2T_ref[...] = pd2T.astype(jnp.bfloat16)
        qd2T_ref[...] = qd2T.astype(jnp.bfloat16)
        w2T_ref[...] = _loop_weight(ps2, qs2, pd2T, qd2T)


def _layer2_kernel(n_ref, adj_ref, ht2_ref, ps2_ref, qs2_ref, pd2T_ref,
                   qd2T_ref, w2T_ref, ht2d_ref, b2_ref,
                   h2_ref, acc_ref):
    i = pl.program_id(0)
    j = pl.program_id(1)
    ns = pl.num_programs(1)
    n = n_ref[0]

    @pl.when(j == 0)
    def _():
        acc_ref[...] = jnp.zeros_like(acc_ref)

    madj = _masked_adj(adj_ref[...], i, j, n)
    e = jnp.maximum(ps2_ref[:, 0:1] * pd2T_ref[0:1, :],
                    qs2_ref[:, 0:1] * qd2T_ref[0:1, :])
    e = e * madj
    acc_ref[...] += jnp.dot(ht2_ref[...], e,
                            preferred_element_type=jnp.float32,
                            precision=_PREC_E)

    @pl.when(j == ns - 1)
    def _():
        acc = acc_ref[...] + w2T_ref[0:1, :] * ht2d_ref[...].astype(jnp.float32)
        norm = acc[0:_NCLASS, :] / acc[_NCLASS:_NCLASS + 1, :]
        h2_ref[...] = norm.T + b2_ref[0:1, :]


@jax.jit
def kernel(adj, x, W1, att_src1, att_dst1, b1, W2, att_src2, att_dst2, b2):
    n = adj.shape[0]
    nt = pl.cdiv(n, _T)
    npad = nt * _T
    heads = _HEADS

    att1s = att_src1[0]
    att1d = att_dst1[0]
    eye = jnp.eye(heads, dtype=jnp.float32)
    asrc_m = (att1s[:, :, None] * eye[:, None, :]).reshape(heads * _NHID,
                                                           heads)
    adst_m = (att1d[:, :, None] * eye[:, None, :]).reshape(heads * _NHID,
                                                           heads)
    as2m = jnp.tile(att_src2[0, 0][:, None], (1, 8))
    ad2m = jnp.tile(att_dst2[0, 0][:, None], (1, 8))
    b1_2d = b1.reshape(1, -1)
    b2_2d = b2.reshape(1, -1)
    n_arr = jnp.full((1,), n, dtype=jnp.int32)

    nfeat = x.shape[1]
    hc = heads * _NHID
    nspec = pl.BlockSpec(memory_space=pltpu.SMEM)

    def nvec(idx, dt=jnp.bfloat16):
        return pl.BlockSpec((_T, 8), idx), jax.ShapeDtypeStruct(
            (npad, 8), dt)

    def lvec(idx, dt=jnp.bfloat16):
        return pl.BlockSpec((8, _T), idx), jax.ShapeDtypeStruct(
            (8, npad), dt)

    p_specs = [
        nvec(lambda k: (k, 0)), nvec(lambda k: (k, 0)),
        lvec(lambda k: (0, k)), lvec(lambda k: (0, k)),
        lvec(lambda k: (0, k), jnp.float32),
    ]
    ht1, ps1, qs1, pd1T, qd1T, w1T = pl.pallas_call(
        _prep_kernel,
        grid=(nt,),
        in_specs=[
            nspec,
            pl.BlockSpec((_T, nfeat), lambda k: (k, 0)),
            pl.BlockSpec((nfeat, hc), lambda k: (0, 0)),
            pl.BlockSpec((hc, heads), lambda k: (0, 0)),
            pl.BlockSpec((hc, heads), lambda k: (0, 0)),
        ],
        out_specs=[pl.BlockSpec((heads, _HP, _T), lambda k: (0, 0, k))] +
                  [s for s, _ in p_specs],
        out_shape=[jax.ShapeDtypeStruct((heads, _HP, npad), jnp.bfloat16)] +
                  [sh for _, sh in p_specs],
    )(n_arr, x, W1, asrc_m, adst_m)

    ts = _TSM * _T

    def nvec_s(idx):
        return pl.BlockSpec((ts, 8), idx)

    q_specs = [
        (nvec_s(lambda i, j: (j, 0)), None),
        (nvec_s(lambda i, j: (j, 0)), None),
        lvec(lambda i, j: (0, i)), lvec(lambda i, j: (0, i)),
        lvec(lambda i, j: (0, i), jnp.float32),
    ]
    nts = nt // _TSM
    ts = _TSM * _T
    h1, ht2, ps2, qs2, pd2T, qd2T, w2T = pl.pallas_call(
        _layer1_kernel,
        grid=(nt, nts),
        in_specs=[
            nspec,
            pl.BlockSpec((ts, _T), lambda i, j: (j, i)),
            pl.BlockSpec((heads, _HP, ts), lambda i, j: (0, 0, j)),
        ] + [s for s, _ in q_specs] + [
            pl.BlockSpec((heads, _HP, _T), lambda i, j: (0, 0, i)),
            pl.BlockSpec((hc, _NCLASS), lambda i, j: (0, 0)),
            pl.BlockSpec((1, hc), lambda i, j: (0, 0)),
            pl.BlockSpec((_NCLASS, 8), lambda i, j: (0, 0)),
            pl.BlockSpec((_NCLASS, 8), lambda i, j: (0, 0)),
        ],
        out_specs=[
            pl.BlockSpec((_T, hc), lambda i, j: (i, 0)),
            pl.BlockSpec((_HP, _T), lambda i, j: (0, i)),
            pl.BlockSpec((_T, 8), lambda i, j: (i, 0)),
            pl.BlockSpec((_T, 8), lambda i, j: (i, 0)),
            pl.BlockSpec((8, _T), lambda i, j: (0, i)),
            pl.BlockSpec((8, _T), lambda i, j: (0, i)),
            pl.BlockSpec((8, _T), lambda i, j: (0, i)),
        ],
        out_shape=[
            jax.ShapeDtypeStruct((n, hc), jnp.float32),
            jax.ShapeDtypeStruct((_HP, npad), jnp.bfloat16),
            jax.ShapeDtypeStruct((npad, 8), jnp.bfloat16),
            jax.ShapeDtypeStruct((npad, 8), jnp.bfloat16),
            jax.ShapeDtypeStruct((8, npad), jnp.bfloat16),
            jax.ShapeDtypeStruct((8, npad), jnp.bfloat16),
            jax.ShapeDtypeStruct((8, npad), jnp.float32),
        ],
        scratch_shapes=[pltpu.VMEM((heads, _HP, _T), jnp.float32)],
    )(n_arr, adj, ht1, ps1, qs1, pd1T, qd1T, w1T, ht1, W2, b1_2d,
      as2m, ad2m)

    h2 = pl.pallas_call(
        _layer2_kernel,
        grid=(nt, nts),
        in_specs=[
            nspec,
            pl.BlockSpec((ts, _T), lambda i, j: (j, i)),
            pl.BlockSpec((_HP, ts), lambda i, j: (0, j)),
            pl.BlockSpec((ts, 8), lambda i, j: (j, 0)),
            pl.BlockSpec((ts, 8), lambda i, j: (j, 0)),
            pl.BlockSpec((8, _T), lambda i, j: (0, i)),
            pl.BlockSpec((8, _T), lambda i, j: (0, i)),
            pl.BlockSpec((8, _T), lambda i, j: (0, i)),
            pl.BlockSpec((_HP, _T), lambda i, j: (0, i)),
            pl.BlockSpec((1, _NCLASS), lambda i, j: (0, 0)),
        ],
        out_specs=pl.BlockSpec((_T, _NCLASS), lambda i, j: (i, 0)),
        out_shape=jax.ShapeDtypeStruct((n, _NCLASS), jnp.float32),
        scratch_shapes=[pltpu.VMEM((_HP, _T), jnp.float32)],
    )(n_arr, adj, ht2, ps2, qs2, pd2T, qd2T, w2T, ht2, b2_2d)

    return (h2, h1, h2)
